# Initial kernel scaffold; baseline (speedup 1.0000x reference)
import jax
import jax.numpy as jnp
from jax.experimental import pallas as pl


def kernel(indices, pe):
    raise NotImplementedError("write your pallas kernel here")


# speedup vs baseline: 3.7343x; 3.7343x over previous
import functools

import jax
import jax.numpy as jnp
from jax import lax
from jax.experimental import pallas as pl
from jax.experimental.pallas import tpu as pltpu
from jax.experimental.pallas import tpu_sc as plsc

D_MODEL = 64

_NC = 2
_NS = 16
_NW = _NC * _NS
_CHUNK = 128


@functools.partial(jax.jit, static_argnums=())
def _flat_gather(table, idx_flat):
    B = idx_flat.shape[0]
    b_per_w = B // _NW
    n_chunks = b_per_w // _CHUNK
    mesh = plsc.VectorSubcoreMesh(core_axis_name="c", subcore_axis_name="s")

    @functools.partial(
        pl.kernel,
        mesh=mesh,
        compiler_params=pltpu.CompilerParams(use_tc_tiling_on_sc=False),
        out_type=jax.ShapeDtypeStruct((B, D_MODEL), jnp.float32),
        scratch_types=[
            pltpu.VMEM((_CHUNK,), jnp.int32),
            pltpu.VMEM((_CHUNK, D_MODEL), jnp.float32),
            pltpu.SemaphoreType.DMA,
        ],
    )
    def k(table_hbm, idx_hbm, out_hbm, idx_v, rows_v, sem):
        wid = lax.axis_index("s") * _NC + lax.axis_index("c")
        base = wid * b_per_w

        def body(i, carry):
            off = base + i * _CHUNK
            pltpu.sync_copy(idx_hbm.at[pl.ds(off, _CHUNK)], idx_v)
            pltpu.async_copy(table_hbm.at[idx_v], rows_v, sem).wait()
            pltpu.sync_copy(rows_v, out_hbm.at[pl.ds(off, _CHUNK)])
            return carry

        lax.fori_loop(0, n_chunks, body, 0)

    return k(table, idx_flat)


def kernel(indices, pe):
    b0, b1 = indices.shape
    flat = indices.reshape(b0 * b1).astype(jnp.int32)
    out = _flat_gather(pe, flat)
    return out.reshape(b0, b1, D_MODEL)


# speedup vs baseline: 4.9454x; 1.3243x over previous
import functools

import jax
import jax.numpy as jnp
from jax import lax
from jax.experimental import pallas as pl
from jax.experimental.pallas import tpu as pltpu
from jax.experimental.pallas import tpu_sc as plsc

D_MODEL = 64

_NC = 2
_NS = 16
_NW = _NC * _NS
_CHUNK = 128
_NB = 8
_GA = 4


def _flat_gather(table, idx_flat):
    B = idx_flat.shape[0]
    b_per_w = B // _NW
    n_chunks = b_per_w // _CHUNK
    mesh = plsc.VectorSubcoreMesh(core_axis_name="c", subcore_axis_name="s")

    @functools.partial(
        pl.kernel,
        mesh=mesh,
        compiler_params=pltpu.CompilerParams(use_tc_tiling_on_sc=False),
        out_type=jax.ShapeDtypeStruct((B, D_MODEL), jnp.float32),
        scratch_types=[
            pltpu.VMEM((b_per_w,), jnp.int32),
            pltpu.VMEM((_NB, _CHUNK, D_MODEL), jnp.float32),
            pltpu.SemaphoreType.DMA((_NB,)),
            pltpu.SemaphoreType.DMA((_NB,)),
        ],
    )
    def k(table_hbm, idx_hbm, out_hbm, idx_v, rows_v, gsem, wsem):
        wid = lax.axis_index("s") * _NC + lax.axis_index("c")
        base = wid * b_per_w
        pltpu.sync_copy(idx_hbm.at[pl.ds(base, b_per_w)], idx_v)

        def issue_gather(chunk, buf):
            pltpu.async_copy(
                table_hbm.at[idx_v.at[pl.ds(chunk * _CHUNK, _CHUNK)]],
                rows_v.at[buf],
                gsem.at[buf],
            )

        for j in range(_GA):
            issue_gather(j, j)

        def body(i, carry):
            b = lax.rem(i, _NB)
            pltpu.make_async_copy(
                table_hbm.at[pl.ds(0, _CHUNK)], rows_v.at[b], gsem.at[b]
            ).wait()
            pltpu.async_copy(
                rows_v.at[b],
                out_hbm.at[pl.ds(base + i * _CHUNK, _CHUNK)],
                wsem.at[b],
            )
            nxt = i + _GA

            @pl.when(nxt < n_chunks)
            def _():
                bn = lax.rem(nxt, _NB)

                @pl.when(nxt >= _NB)
                def _():
                    pltpu.make_async_copy(
                        rows_v.at[bn],
                        out_hbm.at[pl.ds(0, _CHUNK)],
                        wsem.at[bn],
                    ).wait()

                issue_gather(nxt, bn)

            return carry

        lax.fori_loop(0, n_chunks, body, 0)

        for j in range(_NB):
            pltpu.make_async_copy(
                rows_v.at[j], out_hbm.at[pl.ds(0, _CHUNK)], wsem.at[j]
            ).wait()

    return k(table, idx_flat)


def kernel(indices, pe):
    b0, b1 = indices.shape
    flat = indices.reshape(b0 * b1).astype(jnp.int32)
    out = _flat_gather(pe, flat)
    return out.reshape(b0, b1, D_MODEL)
